# baseline (speedup 1.0000x reference)
import jax
import jax.numpy as jnp
from jax.experimental import pallas as pl
from jax.experimental.pallas import tpu as pltpu

B, C, H, W = 8, 3, 1024, 1024
CLAMP_MIN = 1e-8
BH = 128


def _ce_kernel(x_ref, y_ref, w_ref, m_ref, out_ref):
    x0 = x_ref[0, 0, :, :]
    x1 = x_ref[0, 1, :, :]
    x2 = x_ref[0, 2, :, :]
    mx = jnp.maximum(jnp.maximum(x0, x1), x2)
    lse = mx + jnp.log(jnp.exp(x0 - mx) + jnp.exp(x1 - mx) + jnp.exp(x2 - mx))

    y = y_ref[0, :, :]
    x_y = jnp.where(y == 0, x0, jnp.where(y == 1, x1, x2))
    logp_y = jnp.maximum(x_y - lse, jnp.float32(jnp.log(CLAMP_MIN)))

    w0 = w_ref[0, 0]
    w1 = w_ref[0, 1]
    w2 = w_ref[0, 2]
    w_y = jnp.where(y == 0, w0, jnp.where(y == 1, w1, w2))

    ce = -w_y * logp_y * m_ref[0, :, :]
    out_ref[0, 0, :, :] = jnp.sum(ce).reshape(1, 1)


def kernel(x, y, weight, loss_mask):
    grid = (B, H // BH)
    partials = pl.pallas_call(
        _ce_kernel,
        grid=grid,
        in_specs=[
            pl.BlockSpec((1, C, BH, W), lambda i, j: (i, 0, j, 0)),
            pl.BlockSpec((1, BH, W), lambda i, j: (i, j, 0)),
            pl.BlockSpec((1, C), lambda i, j: (0, 0)),
            pl.BlockSpec((1, BH, W), lambda i, j: (i, j, 0)),
        ],
        out_specs=pl.BlockSpec((1, 1, 1, 1), lambda i, j: (i, j, 0, 0)),
        out_shape=jax.ShapeDtypeStruct(grid + (1, 1), jnp.float32),
        compiler_params=pltpu.CompilerParams(
            dimension_semantics=("parallel", "parallel"),
        ),
    )(x, y, weight.reshape(1, C), loss_mask)
    denom = jnp.float32(B * H * W)
    return jnp.sum(partials) / denom


# speedup vs baseline: 1.3245x; 1.3245x over previous
import jax
import jax.numpy as jnp
from jax.experimental import pallas as pl
from jax.experimental.pallas import tpu as pltpu

B, C, H, W = 8, 3, 1024, 1024
CLAMP_MIN = 1e-8
BH = 512


def _ce_kernel(x_ref, y_ref, w_ref, m_ref, out_ref):
    x0 = x_ref[0, 0, :, :]
    x1 = x_ref[0, 1, :, :]
    x2 = x_ref[0, 2, :, :]
    mx = jnp.maximum(jnp.maximum(x0, x1), x2)
    lse = mx + jnp.log(jnp.exp(x0 - mx) + jnp.exp(x1 - mx) + jnp.exp(x2 - mx))

    y = y_ref[0, :, :]
    x_y = jnp.where(y == 0, x0, jnp.where(y == 1, x1, x2))
    logp_y = jnp.maximum(x_y - lse, jnp.float32(jnp.log(CLAMP_MIN)))

    w0 = w_ref[0, 0]
    w1 = w_ref[0, 1]
    w2 = w_ref[0, 2]
    w_y = jnp.where(y == 0, w0, jnp.where(y == 1, w1, w2))

    ce = -w_y * logp_y * m_ref[0, :, :]
    out_ref[0, 0, :, :] = jnp.sum(ce).reshape(1, 1)


def kernel(x, y, weight, loss_mask):
    grid = (B, H // BH)
    partials = pl.pallas_call(
        _ce_kernel,
        grid=grid,
        in_specs=[
            pl.BlockSpec((1, C, BH, W), lambda i, j: (i, 0, j, 0)),
            pl.BlockSpec((1, BH, W), lambda i, j: (i, j, 0)),
            pl.BlockSpec((1, C), lambda i, j: (0, 0)),
            pl.BlockSpec((1, BH, W), lambda i, j: (i, j, 0)),
        ],
        out_specs=pl.BlockSpec((1, 1, 1, 1), lambda i, j: (i, j, 0, 0)),
        out_shape=jax.ShapeDtypeStruct(grid + (1, 1), jnp.float32),
        compiler_params=pltpu.CompilerParams(
            dimension_semantics=("parallel", "parallel"),
        ),
    )(x, y, weight.reshape(1, C), loss_mask)
    denom = jnp.float32(B * H * W)
    return jnp.sum(partials) / denom


# speedup vs baseline: 1.3415x; 1.0129x over previous
import jax
import jax.numpy as jnp
from jax.experimental import pallas as pl
from jax.experimental.pallas import tpu as pltpu

B, C, H, W = 8, 3, 1024, 1024
CLAMP_MIN = 1e-8
BH = 1024


def _ce_kernel(x_ref, y_ref, w_ref, m_ref, out_ref):
    x0 = x_ref[0, 0, :, :]
    x1 = x_ref[0, 1, :, :]
    x2 = x_ref[0, 2, :, :]
    mx = jnp.maximum(jnp.maximum(x0, x1), x2)
    lse = mx + jnp.log(jnp.exp(x0 - mx) + jnp.exp(x1 - mx) + jnp.exp(x2 - mx))

    y = y_ref[0, :, :]
    x_y = jnp.where(y == 0, x0, jnp.where(y == 1, x1, x2))
    logp_y = jnp.maximum(x_y - lse, jnp.float32(jnp.log(CLAMP_MIN)))

    w0 = w_ref[0, 0]
    w1 = w_ref[0, 1]
    w2 = w_ref[0, 2]
    w_y = jnp.where(y == 0, w0, jnp.where(y == 1, w1, w2))

    ce = -w_y * logp_y * m_ref[0, :, :]
    out_ref[0, 0, :, :] = jnp.sum(ce).reshape(1, 1)


def kernel(x, y, weight, loss_mask):
    grid = (B, H // BH)
    partials = pl.pallas_call(
        _ce_kernel,
        grid=grid,
        in_specs=[
            pl.BlockSpec((1, C, BH, W), lambda i, j: (i, 0, j, 0)),
            pl.BlockSpec((1, BH, W), lambda i, j: (i, j, 0)),
            pl.BlockSpec((1, C), lambda i, j: (0, 0)),
            pl.BlockSpec((1, BH, W), lambda i, j: (i, j, 0)),
        ],
        out_specs=pl.BlockSpec((1, 1, 1, 1), lambda i, j: (i, j, 0, 0)),
        out_shape=jax.ShapeDtypeStruct(grid + (1, 1), jnp.float32),
        compiler_params=pltpu.CompilerParams(
            dimension_semantics=("parallel", "parallel"),
        ),
    )(x, y, weight.reshape(1, C), loss_mask)
    denom = jnp.float32(B * H * W)
    return jnp.sum(partials) / denom
